# baseline (speedup 1.0000x reference)
import jax
import jax.numpy as jnp
from jax import lax
from jax.experimental import pallas as pl
from jax.experimental.pallas import tpu as pltpu

B, N, C, T, K = 8, 1024, 512, 8192, 20
KPAD = 32
NEG = -3.0e38
BIG = 3.0e38
_NCHUNK = 4
_CHUNK = T // _NCHUNK
_NSLC = _CHUNK // 128


def _tree(cands):
    while len(cands) > 1:
        nxt = []
        for j in range(0, len(cands) - 1, 2):
            (av, ac), (bv, bc) = cands[j], cands[j + 1]
            t = bv > av
            nxt.append((jnp.where(t, bv, av), jnp.where(t, bc, ac)))
        if len(cands) % 2:
            nxt.append(cands[-1])
        cands = nxt
    return cands[0]


def _topk_body(g_ref, t_ref, idx_ref, lg_ref, av_ref, ac_ref):
    i = pl.program_id(0)
    g = g_ref[:, 0, :]
    tb = t_ref[...]
    lg_ref[i] = lax.dot_general(
        g, tb, (((1,), (1,)), ((), ())), preferred_element_type=jnp.float32
    )

    lane = lax.broadcasted_iota(jnp.int32, (B, 128), 1)

    def _slices(chunks, prev_v, prev_c):
        cands = []
        for c in chunks:
            for s in range(_NSLC):
                v = lg_ref[c, :, s * 128:(s + 1) * 128]
                col = c * _CHUNK + s * 128 + lane
                elig = (v < prev_v) | ((v == prev_v) & (col > prev_c))
                cands.append((jnp.where(elig, v, NEG), col))
        return cands

    @pl.when(i == _NCHUNK - 2)
    def _():
        accv = jnp.full((B, 128), NEG, jnp.float32)
        accc = jnp.full((B, 128), T, jnp.int32)
        pv = jnp.full((B, 1), BIG, jnp.float32)
        pc = jnp.full((B, 1), -1, jnp.int32)
        for k in range(K):
            rv, rc = _tree(_slices(range(_NCHUNK - 1), pv, pc))
            mv = jnp.max(rv, axis=1, keepdims=True)
            mc = jnp.min(
                jnp.where(rv == mv, rc, T), axis=1, keepdims=True
            )
            accv = jnp.where(lane == k, mv, accv)
            accc = jnp.where(lane == k, mc, accc)
            pv, pc = mv, mc
        av_ref[...] = accv
        ac_ref[...] = accc

    @pl.when(i == _NCHUNK - 1)
    def _():
        kl = lax.broadcasted_iota(jnp.int32, (B, KPAD), 1)
        av = av_ref[...]
        ac = ac_ref[...]
        acc = jnp.zeros((B, KPAD), jnp.int32)
        pv = jnp.full((B, 1), BIG, jnp.float32)
        pc = jnp.full((B, 1), -1, jnp.int32)
        for k in range(K):
            ea = (av < pv) | ((av == pv) & (ac > pc))
            cands = [(jnp.where(ea, av, NEG), ac)]
            cands += _slices([_NCHUNK - 1], pv, pc)
            rv, rc = _tree(cands)
            mv = jnp.max(rv, axis=1, keepdims=True)
            mc = jnp.min(
                jnp.where(rv == mv, rc, T), axis=1, keepdims=True
            )
            acc = jnp.where(kl == k, mc, acc)
            pv, pc = mv, mc
        idx_ref[...] = acc


def _topk(g_feat, table):
    return pl.pallas_call(
        _topk_body,
        grid=(_NCHUNK,),
        in_specs=[
            pl.BlockSpec((B, 1, C), lambda i: (0, 0, 0)),
            pl.BlockSpec((_CHUNK, C), lambda i: (i, 0)),
        ],
        out_specs=pl.BlockSpec((B, KPAD), lambda i: (0, 0)),
        out_shape=jax.ShapeDtypeStruct((B, KPAD), jnp.int32),
        scratch_shapes=[
            pltpu.VMEM((_NCHUNK, B, _CHUNK), jnp.float32),
            pltpu.VMEM((B, 128), jnp.float32),
            pltpu.VMEM((B, 128), jnp.int32),
        ],
    )(g_feat, table)


def _agg_body(idx_sref, x_ref, t_ref, o_ref, agg_ref, sem):
    b = pl.program_id(0)

    def row_copy(bb, k):
        row = idx_sref[bb, k]
        return pltpu.make_async_copy(
            t_ref.at[pl.ds(row, 1), :],
            agg_ref.at[bb, pl.ds(k, 1), :],
            sem.at[bb],
        )

    @pl.when(b == 0)
    def _():
        for bb in range(B):
            for k in range(K):
                row_copy(bb, k).start()

    for k in range(K):
        row_copy(b, k).wait()

    a = agg_ref[b]
    x = x_ref[0]
    seg = lax.dot_general(
        a, x, (((1,), (1,)), ((), ())), preferred_element_type=jnp.float32
    )
    ki = lax.broadcasted_iota(jnp.int32, (KPAD, N), 0)
    seg = jnp.where(ki < K, seg, NEG)
    m = jnp.max(seg, axis=0, keepdims=True)
    kst = jnp.min(jnp.where(seg == m, ki, KPAD), axis=0, keepdims=True)
    onehot = (ki == kst).astype(jnp.float32)
    counts = jnp.sum(onehot, axis=1)
    scale = 1.0 / (counts + 1.0)
    sa = a * scale[:, None]
    kr = lax.broadcasted_iota(jnp.int32, (KPAD, C), 0)
    sa = jnp.where(kr < K, sa, 0.0)
    o_ref[0] = lax.dot_general(
        onehot, sa, (((0,), (0,)), ((), ())), preferred_element_type=jnp.float32
    )


def _aggregate(idx, inp, table):
    return pl.pallas_call(
        _agg_body,
        grid_spec=pltpu.PrefetchScalarGridSpec(
            num_scalar_prefetch=1,
            grid=(B,),
            in_specs=[
                pl.BlockSpec((1, N, C), lambda b, idx: (b, 0, 0)),
                pl.BlockSpec(memory_space=pl.ANY),
            ],
            out_specs=pl.BlockSpec((1, N, C), lambda b, idx: (b, 0, 0)),
            scratch_shapes=[
                pltpu.VMEM((B, KPAD, C), jnp.float32),
                pltpu.SemaphoreType.DMA((B,)),
            ],
        ),
        out_shape=jax.ShapeDtypeStruct((B, N, C), jnp.float32),
    )(idx, inp, table)


@jax.jit
def kernel(g_feat, input, tau, text_embeddings):
    del tau
    idx = _topk(g_feat, text_embeddings)
    return _aggregate(idx, input, text_embeddings)


# speedup vs baseline: 1.1406x; 1.1406x over previous
import jax
import jax.numpy as jnp
from jax import lax
from jax.experimental import pallas as pl
from jax.experimental.pallas import tpu as pltpu

B, N, C, T, K = 8, 1024, 512, 8192, 20
KPAD = 32
NEG = -3.0e38
BIG = 3.0e38
_NCHUNK = 4
_CHUNK = T // _NCHUNK
_NSLC = _CHUNK // 128


def _tree(cands):
    while len(cands) > 1:
        nxt = []
        for j in range(0, len(cands) - 1, 2):
            (av, ac), (bv, bc) = cands[j], cands[j + 1]
            t = bv > av
            nxt.append((jnp.where(t, bv, av), jnp.where(t, bc, ac)))
        if len(cands) % 2:
            nxt.append(cands[-1])
        cands = nxt
    return cands[0]


def _topk_body(g_ref, t_ref, idx_ref, lg_ref):
    i = pl.program_id(0)
    g = g_ref[:, 0, :]
    tb = t_ref[...]
    lg_ref[i] = lax.dot_general(
        g, tb, (((1,), (1,)), ((), ())), preferred_element_type=jnp.float32
    )

    @pl.when(i == _NCHUNK - 1)
    def _():
        lane = lax.broadcasted_iota(jnp.int32, (B, 128), 1)
        kl = lax.broadcasted_iota(jnp.int32, (B, KPAD), 1)
        acc = jnp.zeros((B, KPAD), jnp.int32)
        prev_v = jnp.full((B, 1), BIG, jnp.float32)
        prev_c = jnp.full((B, 1), -1, jnp.int32)
        for k in range(K):
            cands = []
            for c in range(_NCHUNK):
                for s in range(_NSLC):
                    v = lg_ref[c, :, s * 128:(s + 1) * 128]
                    col = c * _CHUNK + s * 128 + lane
                    elig = (v < prev_v) | ((v == prev_v) & (col > prev_c))
                    cands.append((jnp.where(elig, v, NEG), col))
            rv, rc = _tree(cands)
            mv = jnp.max(rv, axis=1, keepdims=True)
            mc = jnp.min(
                jnp.where(rv == mv, rc, T), axis=1, keepdims=True
            )
            acc = jnp.where(kl == k, mc, acc)
            prev_v, prev_c = mv, mc
        idx_ref[...] = acc


def _topk(g_feat, table):
    return pl.pallas_call(
        _topk_body,
        grid=(_NCHUNK,),
        in_specs=[
            pl.BlockSpec((B, 1, C), lambda i: (0, 0, 0)),
            pl.BlockSpec((_CHUNK, C), lambda i: (i, 0)),
        ],
        out_specs=pl.BlockSpec((B, KPAD), lambda i: (0, 0)),
        out_shape=jax.ShapeDtypeStruct((B, KPAD), jnp.int32),
        scratch_shapes=[pltpu.VMEM((_NCHUNK, B, _CHUNK), jnp.float32)],
    )(g_feat, table)


def _agg_body(idx_sref, x_ref, t_ref, o_ref, agg_ref, sem):
    step = pl.program_id(0)
    b = jnp.maximum(step - 1, 0)

    def row_copy(bb, k):
        row = idx_sref[bb, k]
        return pltpu.make_async_copy(
            t_ref.at[pl.ds(row, 1), :],
            agg_ref.at[bb, pl.ds(k, 1), :],
            sem.at[bb],
        )

    @pl.when(step == 0)
    def _():
        for bb in range(B):
            for k in range(K):
                row_copy(bb, k).start()

    @pl.when(step > 0)
    def _():
        _agg_compute(idx_sref, x_ref, o_ref, agg_ref, sem, row_copy, b)


def _agg_compute(idx_sref, x_ref, o_ref, agg_ref, sem, row_copy, b):
    for k in range(K):
        row_copy(b, k).wait()

    a = agg_ref[b]
    x = x_ref[0]
    seg = lax.dot_general(
        a, x, (((1,), (1,)), ((), ())), preferred_element_type=jnp.float32
    )
    ki = lax.broadcasted_iota(jnp.int32, (KPAD, N), 0)
    seg = jnp.where(ki < K, seg, NEG)
    m = jnp.max(seg, axis=0, keepdims=True)
    kst = jnp.min(jnp.where(seg == m, ki, KPAD), axis=0, keepdims=True)
    onehot = (ki == kst).astype(jnp.float32)
    counts = jnp.sum(onehot, axis=1)
    scale = 1.0 / (counts + 1.0)
    sa = a * scale[:, None]
    kr = lax.broadcasted_iota(jnp.int32, (KPAD, C), 0)
    sa = jnp.where(kr < K, sa, 0.0)
    o_ref[0] = lax.dot_general(
        onehot, sa, (((0,), (0,)), ((), ())), preferred_element_type=jnp.float32
    )


def _aggregate(idx, inp, table):
    return pl.pallas_call(
        _agg_body,
        grid_spec=pltpu.PrefetchScalarGridSpec(
            num_scalar_prefetch=1,
            grid=(B + 1,),
            in_specs=[
                pl.BlockSpec(
                    (1, N, C), lambda s, idx: (jnp.maximum(s - 1, 0), 0, 0)
                ),
                pl.BlockSpec(memory_space=pl.ANY),
            ],
            out_specs=pl.BlockSpec(
                (1, N, C), lambda s, idx: (jnp.maximum(s - 1, 0), 0, 0)
            ),
            scratch_shapes=[
                pltpu.VMEM((B, KPAD, C), jnp.float32),
                pltpu.SemaphoreType.DMA((B,)),
            ],
        ),
        out_shape=jax.ShapeDtypeStruct((B, N, C), jnp.float32),
    )(idx, inp, table)


@jax.jit
def kernel(g_feat, input, tau, text_embeddings):
    del tau
    idx = _topk(g_feat, text_embeddings)
    return _aggregate(idx, input, text_embeddings)
